# baseline (speedup 1.0000x reference)
import jax
import jax.numpy as jnp
from jax.experimental import pallas as pl
from jax.experimental.pallas import tpu as pltpu


def _decoder_body(inp_ref, hid_ref, mbT_ref, mem_bias_ref, mem_rows_ref,
                  a_row_ref, a_col_ref,
                  W_h2w_ref, W_i2w_ref, W_m2w_ref, W_m2wT_ref, W_u2w_ref,
                  W_h2gates_ref, W_i2gates_ref, W_r2gates_ref,
                  W_h2ab_ref, W_i2ab_ref, W_r2ab_ref,
                  W_h2c_ref, W_i2c_ref, W_r2c_ref, W_h2m_ref,
                  out_ref, logits_scr):
    f32 = jnp.float32
    SEQ, BSZ, _ = inp_ref.shape
    K, N = mbT_ref.shape
    A = K - W_h2m_ref.shape[1]

    def dot(x, y):
        return jax.lax.dot_general(x, y, (((1,), (0,)), ((), ())),
                                   precision=jax.lax.Precision.DEFAULT,
                                   preferred_element_type=f32)

    def bf16r(x):
        return x.astype(jnp.bfloat16).astype(f32)

    base_T = dot(W_m2wT_ref[...], mbT_ref[...])
    a_row = a_row_ref[...]
    a_col = a_col_ref[...]

    h = hid_ref[0]
    c = jnp.zeros_like(h)
    w_sum = jnp.zeros((BSZ, N), f32)
    lane_iota = jax.lax.broadcasted_iota(jnp.int32, (BSZ, N), 1)
    vals = []

    for t in range(SEQ):
        emb = inp_ref[t]
        mu = jnp.mean(w_sum, axis=1, keepdims=True)
        var = jnp.mean((w_sum - mu) ** 2, axis=1, keepdims=True)
        u = (w_sum - mu) / jnp.sqrt(var + 1e-5)
        pvAB = dot(h, W_h2w_ref[...]) + dot(emb, W_i2w_ref[...])
        U = dot(u, W_u2w_ref[...])
        pvT = jnp.transpose(pvAB + U)
        for b in range(BSZ):
            x = base_T + pvT[:, b:b + 1]
            logits_scr[b:b + 1, :] = dot(a_row, jnp.tanh(x))
        logits = logits_scr[...]
        for s in range(len(vals)):
            row_a = jnp.broadcast_to(mem_rows_ref[s:s + 1, :A], (BSZ, A))
            M = jnp.concatenate([row_a, vals[s]], axis=1)
            hp = jnp.tanh((pvAB + dot(M, W_m2w_ref[...])) + U)
            lrow = dot(hp, a_col)
            logits = jnp.where(lane_iota == SEQ * s, lrow, logits)
        m = jnp.max(logits, axis=1, keepdims=True)
        idx = jnp.min(jnp.where(logits == m, lane_iota, N), axis=1,
                      keepdims=True)
        onehot = (lane_iota == idx).astype(f32)
        w_sum = w_sum + onehot
        r = dot(onehot, mem_bias_ref[...])
        anymask = jnp.zeros((BSZ, 1), f32)
        rc = jnp.zeros((BSZ, K - A), f32)
        for s in range(len(vals)):
            mk = (idx == SEQ * s).astype(f32)
            anymask = anymask + mk
            rc = rc + mk * bf16r(vals[s])
        r_c = jnp.where(anymask > 0, rc, r[:, A:])
        r = jnp.concatenate([r[:, :A], r_c], axis=1)
        gates = jax.nn.sigmoid(dot(h, W_h2gates_ref[...])
                               + dot(emb, W_i2gates_ref[...])
                               + dot(r, W_r2gates_ref[...]))
        f = gates[:, 0:1]
        i = gates[:, 1:2]
        o = gates[:, 2:3]
        ab = (dot(h, W_h2ab_ref[...]) + dot(emb, W_i2ab_ref[...])
              + dot(r, W_r2ab_ref[...]))
        alpha = (ab[:, 0:1] > 0).astype(f32)
        beta = (ab[:, 1:2] > 0).astype(f32)
        c_cand = jnp.tanh(beta * dot(h, W_h2c_ref[...])
                          + dot(emb, W_i2c_ref[...])
                          + alpha * dot(r, W_r2c_ref[...]))
        c = f * c + i * c_cand
        h = o * jnp.tanh(c)
        out_ref[t] = h
        vals.append(dot(h, W_h2m_ref[...]))


def _make_call(SEQ, BSZ, IDIM, HDIM, N, K, W_h2m_shape, NCORES=2,
               interpret=False):
    B2 = BSZ // NCORES
    full = lambda *shape: pl.BlockSpec(shape, lambda i: (0,) * len(shape))
    bspec = [
        pl.BlockSpec((SEQ, B2, IDIM), lambda i: (0, i, 0)),
        pl.BlockSpec((1, B2, HDIM), lambda i: (0, i, 0)),
        full(K, N),
        full(N, K),
        full(SEQ, K),
        full(1, K), full(K, 1),
        full(HDIM, K), full(IDIM, K), full(K, K), full(K, K), full(N, K),
        full(HDIM, 3), full(IDIM, 3), full(K, 3),
        full(HDIM, 2), full(IDIM, 2), full(K, 2),
        full(HDIM, HDIM), full(IDIM, HDIM), full(K, HDIM),
        full(*W_h2m_shape),
    ]
    return pl.pallas_call(
        _decoder_body,
        grid=(NCORES,),
        in_specs=bspec,
        out_specs=pl.BlockSpec((SEQ, B2, HDIM), lambda i: (0, i, 0)),
        out_shape=jax.ShapeDtypeStruct((SEQ, BSZ, HDIM), jnp.float32),
        scratch_shapes=[pltpu.VMEM((B2, 4096), jnp.float32)],
        compiler_params=pltpu.CompilerParams(
            dimension_semantics=("parallel",),
            vmem_limit_bytes=100 * 1024 * 1024),
        interpret=interpret,
    )


@jax.jit
def kernel(inp, hid, mem_bias, atten_base, W_h2w, W_i2w, W_m2w, W_u2w,
           W_h2gates, W_i2gates, W_r2gates, W_h2ab, W_i2ab, W_r2ab,
           W_h2c, W_i2c, W_r2c, W_h2tau, b_h2tau, W_h2m):
    SEQ, BSZ, IDIM = inp.shape
    HDIM = hid.shape[2]
    N, K = mem_bias.shape
    mbT = mem_bias.T
    mem_rows = mem_bias[:SEQ * SEQ:SEQ]
    a_row = atten_base[0, :, 0][None, :]
    a_col = atten_base[0]
    W_m2wT = W_m2w.T
    return _make_call(SEQ, BSZ, IDIM, HDIM, N, K, W_h2m.shape)(
        inp, hid, mbT, mem_bias, mem_rows, a_row, a_col,
        W_h2w, W_i2w, W_m2w, W_m2wT, W_u2w,
        W_h2gates, W_i2gates, W_r2gates,
        W_h2ab, W_i2ab, W_r2ab,
        W_h2c, W_i2c, W_r2c, W_h2m)


# speedup vs baseline: 1.0741x; 1.0741x over previous
import jax
import jax.numpy as jnp
from jax.experimental import pallas as pl
from jax.experimental.pallas import tpu as pltpu


def _decoder_body(inp_ref, hid_ref, mbT_ref, mem_bias_ref, mem_rows_ref,
                  a_row_ref, a_col_ref,
                  W_h2w_ref, W_i2w_ref, W_m2w_ref, W_m2wT_ref, W_u2w_ref,
                  W_h2gates_ref, W_i2gates_ref, W_r2gates_ref,
                  W_h2ab_ref, W_i2ab_ref, W_r2ab_ref,
                  W_h2c_ref, W_i2c_ref, W_r2c_ref, W_h2m_ref, a_blk_ref,
                  out_ref, logits_scr):
    f32 = jnp.float32
    SEQ, BSZ, _ = inp_ref.shape
    K, N = mbT_ref.shape
    A = K - W_h2m_ref.shape[1]

    def dot(x, y):
        return jax.lax.dot_general(x, y, (((1,), (0,)), ((), ())),
                                   precision=jax.lax.Precision.DEFAULT,
                                   preferred_element_type=f32)

    def bf16r(x):
        return x.astype(jnp.bfloat16).astype(f32)

    base_T = dot(W_m2wT_ref[...], mbT_ref[...])
    a_col = a_col_ref[...]
    G = a_blk_ref.shape[0]

    h = hid_ref[0]
    c = jnp.zeros_like(h)
    w_sum = jnp.zeros((BSZ, N), f32)
    lane_iota = jax.lax.broadcasted_iota(jnp.int32, (BSZ, N), 1)
    vals = []

    for t in range(SEQ):
        emb = inp_ref[t]
        mu = jnp.mean(w_sum, axis=1, keepdims=True)
        var = jnp.mean((w_sum - mu) ** 2, axis=1, keepdims=True)
        u = (w_sum - mu) / jnp.sqrt(var + 1e-5)
        pvAB = dot(h, W_h2w_ref[...]) + dot(emb, W_i2w_ref[...])
        U = dot(u, W_u2w_ref[...])
        pvT = jnp.transpose(pvAB + U)
        for g in range(BSZ // G):
            x = jnp.concatenate(
                [base_T + pvT[:, b:b + 1]
                 for b in range(g * G, (g + 1) * G)], axis=0)
            logits_scr[g * G:(g + 1) * G, :] = dot(a_blk_ref[...],
                                                   jnp.tanh(x))
        logits = logits_scr[...]
        for s in range(len(vals)):
            row_a = jnp.broadcast_to(mem_rows_ref[s:s + 1, :A], (BSZ, A))
            M = jnp.concatenate([row_a, vals[s]], axis=1)
            hp = jnp.tanh((pvAB + dot(M, W_m2w_ref[...])) + U)
            lrow = dot(hp, a_col)
            logits = jnp.where(lane_iota == SEQ * s, lrow, logits)
        m = jnp.max(logits, axis=1, keepdims=True)
        idx = jnp.min(jnp.where(logits == m, lane_iota, N), axis=1,
                      keepdims=True)
        onehot = (lane_iota == idx).astype(f32)
        w_sum = w_sum + onehot
        r = dot(onehot, mem_bias_ref[...])
        anymask = jnp.zeros((BSZ, 1), f32)
        rc = jnp.zeros((BSZ, K - A), f32)
        for s in range(len(vals)):
            mk = (idx == SEQ * s).astype(f32)
            anymask = anymask + mk
            rc = rc + mk * bf16r(vals[s])
        r_c = jnp.where(anymask > 0, rc, r[:, A:])
        r = jnp.concatenate([r[:, :A], r_c], axis=1)
        gates = jax.nn.sigmoid(dot(h, W_h2gates_ref[...])
                               + dot(emb, W_i2gates_ref[...])
                               + dot(r, W_r2gates_ref[...]))
        f = gates[:, 0:1]
        i = gates[:, 1:2]
        o = gates[:, 2:3]
        ab = (dot(h, W_h2ab_ref[...]) + dot(emb, W_i2ab_ref[...])
              + dot(r, W_r2ab_ref[...]))
        alpha = (ab[:, 0:1] > 0).astype(f32)
        beta = (ab[:, 1:2] > 0).astype(f32)
        c_cand = jnp.tanh(beta * dot(h, W_h2c_ref[...])
                          + dot(emb, W_i2c_ref[...])
                          + alpha * dot(r, W_r2c_ref[...]))
        c = f * c + i * c_cand
        h = o * jnp.tanh(c)
        out_ref[t] = h
        vals.append(dot(h, W_h2m_ref[...]))


def _make_call(SEQ, BSZ, HDIM, interpret=False):
    return pl.pallas_call(
        _decoder_body,
        out_shape=jax.ShapeDtypeStruct((SEQ, BSZ, HDIM), jnp.float32),
        scratch_shapes=[pltpu.VMEM((BSZ, 4096), jnp.float32)],
        compiler_params=pltpu.CompilerParams(
            vmem_limit_bytes=100 * 1024 * 1024),
        interpret=interpret,
    )


@jax.jit
def kernel(inp, hid, mem_bias, atten_base, W_h2w, W_i2w, W_m2w, W_u2w,
           W_h2gates, W_i2gates, W_r2gates, W_h2ab, W_i2ab, W_r2ab,
           W_h2c, W_i2c, W_r2c, W_h2tau, b_h2tau, W_h2m):
    SEQ, BSZ, _ = inp.shape
    HDIM = hid.shape[2]
    mbT = mem_bias.T
    mem_rows = mem_bias[:SEQ * SEQ:SEQ]
    a_row = atten_base[0, :, 0][None, :]
    a_col = atten_base[0]
    W_m2wT = W_m2w.T
    G = 8
    a_blk = jnp.kron(jnp.eye(G, dtype=jnp.float32), a_row)
    return _make_call(SEQ, BSZ, HDIM)(
        inp, hid, mbT, mem_bias, mem_rows, a_row, a_col,
        W_h2w, W_i2w, W_m2w, W_m2wT, W_u2w,
        W_h2gates, W_i2gates, W_r2gates,
        W_h2ab, W_i2ab, W_r2ab,
        W_h2c, W_i2c, W_r2c, W_h2m, a_blk)


# speedup vs baseline: 1.0749x; 1.0007x over previous
import jax
import jax.numpy as jnp
from jax.experimental import pallas as pl
from jax.experimental.pallas import tpu as pltpu


def _decoder_body(inp_ref, hid_ref, mbT_ref, mem_bias_ref, mem_rows_ref,
                  a_row_ref, a_col_ref,
                  W_h2w_ref, W_i2w_ref, W_m2w_ref, W_m2wT_ref, W_u2w_ref,
                  W_h2gates_ref, W_i2gates_ref, W_r2gates_ref,
                  W_h2ab_ref, W_i2ab_ref, W_r2ab_ref,
                  W_h2c_ref, W_i2c_ref, W_r2c_ref, W_h2m_ref,
                  out_ref, logits_scr):
    f32 = jnp.float32
    SEQ, BSZ, _ = inp_ref.shape
    K, N = mbT_ref.shape
    A = K - W_h2m_ref.shape[1]

    def dot(x, y):
        return jax.lax.dot_general(x, y, (((1,), (0,)), ((), ())),
                                   precision=jax.lax.Precision.DEFAULT,
                                   preferred_element_type=f32)

    def bf16r(x):
        return x.astype(jnp.bfloat16).astype(f32)

    base_T = dot(W_m2wT_ref[...], mbT_ref[...])
    a_row = a_row_ref[...]
    a_col = a_col_ref[...]

    h = hid_ref[0]
    c = jnp.zeros_like(h)
    w_sum = jnp.zeros((BSZ, N), f32)
    lane_iota = jax.lax.broadcasted_iota(jnp.int32, (BSZ, N), 1)
    vals = []

    for t in range(SEQ):
        emb = inp_ref[t]
        mu = jnp.mean(w_sum, axis=1, keepdims=True)
        var = jnp.mean((w_sum - mu) ** 2, axis=1, keepdims=True)
        u = (w_sum - mu) / jnp.sqrt(var + 1e-5)
        pvAB = dot(h, W_h2w_ref[...]) + dot(emb, W_i2w_ref[...])
        U = dot(u, W_u2w_ref[...])
        pvT = jnp.transpose(pvAB + U)
        for b in range(BSZ):
            x = base_T + pvT[:, b:b + 1]
            logits_scr[b:b + 1, :] = dot(a_row, jnp.tanh(x))
        logits = logits_scr[...]
        for s in range(len(vals)):
            row_a = jnp.broadcast_to(mem_rows_ref[s:s + 1, :A], (BSZ, A))
            M = jnp.concatenate([row_a, vals[s]], axis=1)
            hp = jnp.tanh((pvAB + dot(M, W_m2w_ref[...])) + U)
            lrow = dot(hp, a_col)
            logits = jnp.where(lane_iota == SEQ * s, lrow, logits)
        m = jnp.max(logits, axis=1, keepdims=True)
        idx = jnp.min(jnp.where(logits == m, lane_iota, N), axis=1,
                      keepdims=True)
        onehot = (lane_iota == idx).astype(f32)
        w_sum = w_sum + onehot
        r = dot(onehot, mem_bias_ref[...])
        anymask = jnp.zeros((BSZ, 1), f32)
        rc = jnp.zeros((BSZ, K - A), f32)
        for s in range(len(vals)):
            mk = (idx == SEQ * s).astype(f32)
            anymask = anymask + mk
            rc = rc + mk * bf16r(vals[s])
        r_c = jnp.where(anymask > 0, rc, r[:, A:])
        r = jnp.concatenate([r[:, :A], r_c], axis=1)
        gates = jax.nn.sigmoid(dot(h, W_h2gates_ref[...])
                               + dot(emb, W_i2gates_ref[...])
                               + dot(r, W_r2gates_ref[...]))
        f = gates[:, 0:1]
        i = gates[:, 1:2]
        o = gates[:, 2:3]
        ab = (dot(h, W_h2ab_ref[...]) + dot(emb, W_i2ab_ref[...])
              + dot(r, W_r2ab_ref[...]))
        alpha = (ab[:, 0:1] > 0).astype(f32)
        beta = (ab[:, 1:2] > 0).astype(f32)
        c_cand = jnp.tanh(beta * dot(h, W_h2c_ref[...])
                          + dot(emb, W_i2c_ref[...])
                          + alpha * dot(r, W_r2c_ref[...]))
        c = f * c + i * c_cand
        h = o * jnp.tanh(c)
        out_ref[t] = h
        vals.append(dot(h, W_h2m_ref[...]))


def _make_call(SEQ, BSZ, HDIM, interpret=False):
    return pl.pallas_call(
        _decoder_body,
        out_shape=jax.ShapeDtypeStruct((SEQ, BSZ, HDIM), jnp.float32),
        scratch_shapes=[pltpu.VMEM((BSZ, 4096), jnp.float32)],
        compiler_params=pltpu.CompilerParams(
            vmem_limit_bytes=100 * 1024 * 1024),
        interpret=interpret,
    )


@jax.jit
def kernel(inp, hid, mem_bias, atten_base, W_h2w, W_i2w, W_m2w, W_u2w,
           W_h2gates, W_i2gates, W_r2gates, W_h2ab, W_i2ab, W_r2ab,
           W_h2c, W_i2c, W_r2c, W_h2tau, b_h2tau, W_h2m):
    SEQ, BSZ, _ = inp.shape
    HDIM = hid.shape[2]
    mbT = mem_bias.T
    mem_rows = mem_bias[:SEQ * SEQ:SEQ]
    a_row = atten_base[0, :, 0][None, :]
    a_col = atten_base[0]
    W_m2wT = W_m2w.T
    return _make_call(SEQ, BSZ, HDIM)(
        inp, hid, mbT, mem_bias, mem_rows, a_row, a_col,
        W_h2w, W_i2w, W_m2w, W_m2wT, W_u2w,
        W_h2gates, W_i2gates, W_r2gates,
        W_h2ab, W_i2ab, W_r2ab,
        W_h2c, W_i2c, W_r2c, W_h2m)
